# baseline (speedup 1.0000x reference)
import jax
import jax.numpy as jnp
from jax.experimental import pallas as pl

_VOCAB = 1000
_ROWS = 32


def _onehot_block(x_ref, o_ref):
    ids = x_ref[...]
    iota = jax.lax.broadcasted_iota(jnp.int32, (_ROWS, x_ref.shape[1], _VOCAB), 2)
    o_ref[...] = (ids[:, :, None] == iota).astype(jnp.float32)


def kernel(x):
    x = x.astype(jnp.int32)
    B, S = x.shape
    return pl.pallas_call(
        _onehot_block,
        grid=(B // _ROWS,),
        in_specs=[pl.BlockSpec((_ROWS, S), lambda i: (i, 0))],
        out_specs=pl.BlockSpec((_ROWS, S, _VOCAB), lambda i: (i, 0, 0)),
        out_shape=jax.ShapeDtypeStruct((B, S, _VOCAB), jnp.float32),
    )(x)
